# baseline (speedup 1.0000x reference)
import functools

import jax
import jax.numpy as jnp
from jax import lax
from jax.experimental import pallas as pl
from jax.experimental.pallas import tpu as pltpu
from jax.experimental.pallas import tpu_sc as plsc

N_NODES = 10000
N_EDGES = 320000
F = 128
G = 16

NC = 2
NS = 16
NW = NC * NS
EDGES_PER_W = N_EDGES // NW
CHUNK = 80
NCHUNKS = EDGES_PER_W // CHUNK
NRING = 4
NMAIN = NCHUNKS - 1
ROWS_PER_TILE = 624
ROWS_TAIL = N_NODES - NS * ROWS_PER_TILE
NVEC = F // 16


def _bcast_lane(v, i):
    idx = lax.full((16, 1), i, jnp.int32)
    return lax.gather(
        v, idx,
        lax.GatherDimensionNumbers(
            offset_dims=(), collapsed_slice_dims=(0,), start_index_map=(0,)),
        (1,), mode=lax.GatherScatterMode.PROMISE_IN_BOUNDS)


def _sc_agg_body(x_hbm, pk_hbm, out_hbm,
                 acc, srcv, dstv, attrv, rows, sem_g, sem_s, sem_ia, sem_id):
    c = lax.axis_index("c")
    s = lax.axis_index("s")
    wid = s * NC + c
    ebase = wid * EDGES_PER_W

    def _zero_row(r, carry):
        for j in range(NVEC):
            rows[0][r, pl.ds(j * 16, 16)] = jnp.zeros((16,), jnp.float32)
        return carry
    lax.fori_loop(0, CHUNK, _zero_row, 0)
    rbase = s * ROWS_PER_TILE
    for t in range(ROWS_PER_TILE // CHUNK):
        pltpu.sync_copy(rows[0], acc.at[pl.ds(rbase + t * CHUNK, CHUNK)])
    _rem = ROWS_PER_TILE % CHUNK
    if _rem:
        pltpu.sync_copy(
            rows[0].at[pl.ds(0, _rem)],
            acc.at[pl.ds(rbase + (ROWS_PER_TILE // CHUNK) * CHUNK, _rem)])

    @pl.when(s == 0)
    def _zero_tail():
        pltpu.sync_copy(rows[0].at[pl.ds(0, ROWS_TAIL)],
                        acc.at[pl.ds(NS * ROWS_PER_TILE, ROWS_TAIL)])
    plsc.subcore_barrier()

    def issue_ia(k, b):
        base = ebase + k * CHUNK
        pltpu.async_copy(pk_hbm.at[pl.ds(base, CHUNK)], srcv[b], sem_ia[b])
        pltpu.async_copy(pk_hbm.at[pl.ds(2 * N_EDGES + base, CHUNK)],
                         attrv[b], sem_ia[b])

    def wait_ia(b):
        pltpu.make_async_copy(pk_hbm.at[pl.ds(0, CHUNK)], srcv[b],
                              sem_ia[b]).wait()
        pltpu.make_async_copy(pk_hbm.at[pl.ds(0, CHUNK)], attrv[b],
                              sem_ia[b]).wait()

    def issue_id(k, b):
        base = ebase + k * CHUNK
        pltpu.async_copy(pk_hbm.at[pl.ds(N_EDGES + base, CHUNK)],
                         dstv[b], sem_id[b])

    def wait_id(b):
        pltpu.make_async_copy(pk_hbm.at[pl.ds(0, CHUNK)], dstv[b],
                              sem_id[b]).wait()

    def issue_gather(b):
        pltpu.async_copy(x_hbm.at[srcv[b]], rows[b], sem_g[b])

    def wait_gather(b):
        pltpu.make_async_copy(x_hbm.at[srcv[b]], rows[b], sem_g[b]).wait()

    def issue_scatter(b):
        pltpu.async_copy(rows[b], acc.at[dstv[b]], sem_s[b], add=True)

    def wait_scatter(b):
        pltpu.make_async_copy(rows[b], acc.at[dstv[b]], sem_s[b]).wait()

    def multiply(b):
        def _group(g, carry):
            av = lax.bitcast_convert_type(attrv[b][pl.ds(g * 16, 16)],
                                          jnp.float32)
            for i in range(16):
                sc = _bcast_lane(av, i)
                e = g * 16 + i
                for j in range(NVEC):
                    sl = pl.ds(j * 16, 16)
                    rows[b][e, sl] = rows[b][e, sl] * sc
            return carry
        lax.fori_loop(0, CHUNK // 16, _group, 0)

    issue_ia(0, 0)
    issue_ia(1, 1)
    issue_ia(2, 2)
    issue_id(0, 0)
    wait_ia(0)
    issue_gather(0)
    wait_ia(1)
    issue_gather(1)

    def _super(it, carry):
        for r in range(NRING):
            k = it * NRING + r
            p = r
            @pl.when(k >= 2)
            def _w():
                wait_scatter((r + 2) % NRING)
            @pl.when(k + 3 < NCHUNKS)
            def _b1():
                issue_ia(k + 3, (r + 3) % NRING)
            @pl.when(k + 1 < NCHUNKS)
            def _b2():
                issue_id(k + 1, (r + 1) % NRING)
            wait_gather(p)
            multiply(p)
            wait_id(p)
            issue_scatter(p)
            @pl.when(k + 2 < NCHUNKS)
            def _f():
                wait_ia((r + 2) % NRING)
                issue_gather((r + 2) % NRING)
        return carry
    lax.fori_loop(0, NMAIN // NRING, _super, 0)

    wait_scatter(2)
    wait_gather(0)
    multiply(0)
    wait_id(0)
    issue_scatter(0)
    wait_scatter(3)
    wait_scatter(0)

    plsc.subcore_barrier()
    pltpu.sync_copy(acc.at[pl.ds(rbase, ROWS_PER_TILE)],
                    out_hbm.at[c, pl.ds(rbase, ROWS_PER_TILE)])

    @pl.when(s == 0)
    def _out_tail():
        pltpu.sync_copy(acc.at[pl.ds(NS * ROWS_PER_TILE, ROWS_TAIL)],
                        out_hbm.at[c, pl.ds(NS * ROWS_PER_TILE, ROWS_TAIL)])


@jax.jit
def _sc_agg(x, pk):
    mesh = plsc.VectorSubcoreMesh(core_axis_name="c", subcore_axis_name="s")
    return pl.kernel(
        _sc_agg_body,
        out_type=jax.ShapeDtypeStruct((NC, N_NODES, F), jnp.float32),
        mesh=mesh,
        scratch_types=[
            pltpu.VMEM_SHARED((N_NODES, F), jnp.float32),
            [pltpu.VMEM((CHUNK,), jnp.int32) for _ in range(NRING)],
            [pltpu.VMEM((CHUNK,), jnp.int32) for _ in range(NRING)],
            [pltpu.VMEM((CHUNK,), jnp.int32) for _ in range(NRING)],
            [pltpu.VMEM((CHUNK, F), jnp.float32) for _ in range(NRING)],
            [pltpu.SemaphoreType.DMA for _ in range(NRING)],
            [pltpu.SemaphoreType.DMA for _ in range(NRING)],
            [pltpu.SemaphoreType.DMA for _ in range(NRING)],
            [pltpu.SemaphoreType.DMA for _ in range(NRING)],
        ],
    )(x, pk)


ROWBLK = 1000
NBLK = N_NODES // ROWBLK

_DN1 = (((1,), (1,)), ((), ()))


def _tc_body(agg_ref, b_ref, u_ref, wk, wq, bk, bq, k_ref, q_ref):
    xa = agg_ref[0] + agg_ref[1]
    oh = (b_ref[...] == lax.broadcasted_iota(jnp.int32, (ROWBLK, G), 1)
          ).astype(jnp.float32)
    hp = lax.Precision.HIGHEST
    ub = jnp.dot(oh, u_ref[...], precision=hp)
    k_ref[...] = (lax.dot_general(xa, wk[:, :F], _DN1, precision=hp)
                  + lax.dot_general(ub, wk[:, F:], _DN1, precision=hp)
                  + bk[...])
    q_ref[...] = (lax.dot_general(xa, wq[:, :F], _DN1, precision=hp)
                  + lax.dot_general(ub, wq[:, F:], _DN1, precision=hp)
                  + bq[...])


@jax.jit
def _tc_linear(agg, batch2d, u, wk, wq, bk, bq):
    full = lambda *shape: pl.BlockSpec(shape, lambda i: tuple(0 for _ in shape))
    return pl.pallas_call(
        _tc_body,
        grid=(NBLK,),
        in_specs=[
            pl.BlockSpec((NC, ROWBLK, F), lambda i: (0, i, 0)),
            pl.BlockSpec((ROWBLK, 1), lambda i: (i, 0)),
            full(G, F),
            full(F, 2 * F), full(F, 2 * F),
            full(1, F), full(1, F),
        ],
        out_specs=[
            pl.BlockSpec((ROWBLK, F), lambda i: (i, 0)),
            pl.BlockSpec((ROWBLK, F), lambda i: (i, 0)),
        ],
        out_shape=[
            jax.ShapeDtypeStruct((N_NODES, F), jnp.float32),
            jax.ShapeDtypeStruct((N_NODES, F), jnp.float32),
        ],
    )(agg, batch2d, u, wk, wq, bk, bq)


def kernel(x, edge_index, edge_attr, u, batch, WK, bK, WQ, bQ):
    src = edge_index[0].astype(jnp.int32)
    dst = edge_index[1].astype(jnp.int32)
    attr_bits = lax.bitcast_convert_type(edge_attr.astype(jnp.float32),
                                         jnp.int32).reshape(N_EDGES)
    pk = jnp.concatenate([src, dst, attr_bits])
    agg = _sc_agg(x, pk)
    batch2d = batch.astype(jnp.int32).reshape(N_NODES, 1)
    K, Q = _tc_linear(agg, batch2d, u, WK, WQ,
                      bK.reshape(1, F), bQ.reshape(1, F))
    return K, Q


# speedup vs baseline: 1.2524x; 1.2524x over previous
import functools

import jax
import jax.numpy as jnp
from jax import lax
from jax.experimental import pallas as pl
from jax.experimental.pallas import tpu as pltpu
from jax.experimental.pallas import tpu_sc as plsc

N_NODES = 10000
N_EDGES = 320000
F = 128
G = 16

NC = 2
NS = 16
NW = NC * NS
EDGES_PER_W = N_EDGES // NW
CHUNK = 80
NCHUNKS = EDGES_PER_W // CHUNK
NRING = 4
NMAIN = NCHUNKS - 1
ROWS_PER_TILE = 624
ROWS_TAIL = N_NODES - NS * ROWS_PER_TILE
NVEC = F // 16


def _bcast_lane(v, i):
    idx = lax.full((16, 1), i, jnp.int32)
    return lax.gather(
        v, idx,
        lax.GatherDimensionNumbers(
            offset_dims=(), collapsed_slice_dims=(0,), start_index_map=(0,)),
        (1,), mode=lax.GatherScatterMode.PROMISE_IN_BOUNDS)


def _sc_agg_body(x_hbm, ei_hbm, attr_hbm, out_hbm,
                 acc, srcv, dstv, attrv, rows,
                 sem_g, sem_s, sem_ia, sem_id):
    c = lax.axis_index("c")
    s = lax.axis_index("s")
    wid = s * NC + c
    ebase = wid * EDGES_PER_W

    rbase = s * ROWS_PER_TILE
    def issue_ia(k, b):
        base = ebase + k * CHUNK
        pltpu.async_copy(ei_hbm.at[pl.ds(base, CHUNK)], srcv[b], sem_ia[b])
        pltpu.async_copy(attr_hbm.at[pl.ds(base, CHUNK)], attrv[b], sem_ia[b])

    def wait_ia(b):
        pltpu.make_async_copy(ei_hbm.at[pl.ds(0, CHUNK)], srcv[b],
                              sem_ia[b]).wait()
        pltpu.make_async_copy(attr_hbm.at[pl.ds(0, CHUNK)], attrv[b],
                              sem_ia[b]).wait()

    def issue_id(k, b):
        base = ebase + k * CHUNK
        pltpu.async_copy(ei_hbm.at[pl.ds(N_EDGES + base, CHUNK)],
                         dstv[b], sem_id[b])

    def wait_id(b):
        pltpu.make_async_copy(ei_hbm.at[pl.ds(0, CHUNK)], dstv[b],
                              sem_id[b]).wait()

    def issue_gather(b):
        pltpu.async_copy(x_hbm.at[srcv[b]], rows[b], sem_g[b])

    def wait_gather(b):
        pltpu.make_async_copy(x_hbm.at[srcv[b]], rows[b], sem_g[b]).wait()

    def issue_scatter(b):
        pltpu.async_copy(rows[b], acc.at[dstv[b]], sem_s[b], add=True)

    def wait_scatter(b):
        pltpu.make_async_copy(rows[b], acc.at[dstv[b]], sem_s[b]).wait()

    def multiply(b):
        def _group(g, carry):
            av = attrv[b][pl.ds(g * 16, 16)]
            for i in range(16):
                sc = _bcast_lane(av, i)
                e = g * 16 + i
                for j in range(NVEC):
                    sl = pl.ds(j * 16, 16)
                    rows[b][e, sl] = rows[b][e, sl] * sc
            return carry
        lax.fori_loop(0, CHUNK // 16, _group, 0)

    issue_ia(0, 0)
    issue_ia(1, 1)
    issue_ia(2, 2)
    issue_id(0, 0)

    def _zero_row(r, carry):
        for j in range(NVEC):
            rows[3][r, pl.ds(j * 16, 16)] = jnp.zeros((16,), jnp.float32)
        return carry
    lax.fori_loop(0, CHUNK, _zero_row, 0)
    for t in range(ROWS_PER_TILE // CHUNK):
        pltpu.sync_copy(rows[3], acc.at[pl.ds(rbase + t * CHUNK, CHUNK)])
    _rem = ROWS_PER_TILE % CHUNK
    if _rem:
        pltpu.sync_copy(
            rows[3].at[pl.ds(0, _rem)],
            acc.at[pl.ds(rbase + (ROWS_PER_TILE // CHUNK) * CHUNK, _rem)])

    @pl.when(s == 0)
    def _zero_tail():
        pltpu.sync_copy(rows[3].at[pl.ds(0, ROWS_TAIL)],
                        acc.at[pl.ds(NS * ROWS_PER_TILE, ROWS_TAIL)])

    wait_ia(0)
    issue_gather(0)
    wait_ia(1)
    issue_gather(1)
    plsc.subcore_barrier()

    def _super(it, carry):
        for r in range(NRING):
            k = it * NRING + r
            p = r
            @pl.when(k >= 2)
            def _w():
                wait_scatter((r + 2) % NRING)
            @pl.when(k + 3 < NCHUNKS)
            def _b1():
                issue_ia(k + 3, (r + 3) % NRING)
            @pl.when(k + 1 < NCHUNKS)
            def _b2():
                issue_id(k + 1, (r + 1) % NRING)
            @pl.when(k + 2 < NCHUNKS)
            def _f():
                wait_ia((r + 2) % NRING)
                issue_gather((r + 2) % NRING)
            wait_gather(p)
            multiply(p)
            wait_id(p)
            issue_scatter(p)
        return carry
    lax.fori_loop(0, NMAIN // NRING, _super, 0)

    wait_scatter(2)
    wait_gather(0)
    multiply(0)
    wait_id(0)
    issue_scatter(0)
    wait_scatter(3)
    wait_scatter(0)

    plsc.subcore_barrier()
    pltpu.sync_copy(acc.at[pl.ds(rbase, ROWS_PER_TILE)],
                    out_hbm.at[c, pl.ds(rbase, ROWS_PER_TILE)])

    @pl.when(s == 0)
    def _out_tail():
        pltpu.sync_copy(acc.at[pl.ds(NS * ROWS_PER_TILE, ROWS_TAIL)],
                        out_hbm.at[c, pl.ds(NS * ROWS_PER_TILE, ROWS_TAIL)])


@jax.jit
def _sc_agg(x, ei, attr):
    mesh = plsc.VectorSubcoreMesh(core_axis_name="c", subcore_axis_name="s")
    return pl.kernel(
        _sc_agg_body,
        out_type=jax.ShapeDtypeStruct((NC, N_NODES, F), jnp.float32),
        mesh=mesh,
        scratch_types=[
            pltpu.VMEM_SHARED((N_NODES, F), jnp.float32),
            [pltpu.VMEM((CHUNK,), jnp.int32) for _ in range(NRING)],
            [pltpu.VMEM((CHUNK,), jnp.int32) for _ in range(NRING)],
            [pltpu.VMEM((CHUNK,), jnp.float32) for _ in range(NRING)],
            [pltpu.VMEM((CHUNK, F), jnp.float32) for _ in range(NRING)],
            [pltpu.SemaphoreType.DMA for _ in range(NRING)],
            [pltpu.SemaphoreType.DMA for _ in range(NRING)],
            [pltpu.SemaphoreType.DMA for _ in range(NRING)],
            [pltpu.SemaphoreType.DMA for _ in range(NRING)],
        ],
    )(x, ei, attr)


ROWBLK = 400
NBLK = N_NODES // ROWBLK


def _tc_body(agg_ref, b_ref, u_ref, wt, b2, k_ref, q_ref):
    xa = agg_ref[0] + agg_ref[1]
    oh = (b_ref[...] == lax.broadcasted_iota(jnp.int32, (ROWBLK, G), 1)
          ).astype(jnp.float32)
    hp = lax.Precision.HIGHEST
    uw = jnp.dot(u_ref[...], wt[F:, :], precision=hp)
    kq = (jnp.dot(xa, wt[:F, :], precision=hp)
          + jnp.dot(oh, uw, precision=hp) + b2[...])
    k_ref[...] = kq[:, :F]
    q_ref[...] = kq[:, F:]


@jax.jit
def _tc_linear(agg, batch2d, u, wt, b2):
    full = lambda *shape: pl.BlockSpec(shape, lambda i: tuple(0 for _ in shape))
    return pl.pallas_call(
        _tc_body,
        grid=(NBLK,),
        in_specs=[
            pl.BlockSpec((NC, ROWBLK, F), lambda i: (0, i, 0)),
            pl.BlockSpec((ROWBLK, 1), lambda i: (i, 0)),
            full(G, F),
            full(2 * F, 2 * F),
            full(1, 2 * F),
        ],
        out_specs=[
            pl.BlockSpec((ROWBLK, F), lambda i: (i, 0)),
            pl.BlockSpec((ROWBLK, F), lambda i: (i, 0)),
        ],
        out_shape=[
            jax.ShapeDtypeStruct((N_NODES, F), jnp.float32),
            jax.ShapeDtypeStruct((N_NODES, F), jnp.float32),
        ],
    )(agg, batch2d, u, wt, b2)


def kernel(x, edge_index, edge_attr, u, batch, WK, bK, WQ, bQ):
    ei = edge_index.astype(jnp.int32).reshape(2 * N_EDGES)
    attr = edge_attr[:, 0]
    agg = _sc_agg(x, ei, attr)
    batch2d = batch.astype(jnp.int32).reshape(N_NODES, 1)
    wt = jnp.concatenate([WK, WQ], axis=0).T
    b2 = jnp.concatenate([bK, bQ]).reshape(1, 2 * F)
    K, Q = _tc_linear(agg, batch2d, u, wt, b2)
    return K, Q


# speedup vs baseline: 1.3255x; 1.0584x over previous
import functools

import jax
import jax.numpy as jnp
from jax import lax
from jax.experimental import pallas as pl
from jax.experimental.pallas import tpu as pltpu
from jax.experimental.pallas import tpu_sc as plsc

N_NODES = 10000
N_EDGES = 320000
F = 128
G = 16

NC = 2
NS = 16
NW = NC * NS
EDGES_PER_W = N_EDGES // NW
CHUNK = 80
NCHUNKS = EDGES_PER_W // CHUNK
NRING = 4
NMAIN = NCHUNKS - 1
ROWS_PER_TILE = 624
ROWS_TAIL = N_NODES - NS * ROWS_PER_TILE
NVEC = F // 16


def _bcast_lane(v, i):
    idx = lax.full((16, 1), i, jnp.int32)
    return lax.gather(
        v, idx,
        lax.GatherDimensionNumbers(
            offset_dims=(), collapsed_slice_dims=(0,), start_index_map=(0,)),
        (1,), mode=lax.GatherScatterMode.PROMISE_IN_BOUNDS)


def _sc_agg_body(x_hbm, ei_hbm, attr_hbm, out_hbm,
                 acc, srcv, dstv, attrv, rows,
                 sem_g, sem_s, sem_ia, sem_id):
    c = lax.axis_index("c")
    s = lax.axis_index("s")
    wid = s * NC + c
    ebase = wid * EDGES_PER_W

    rbase = s * ROWS_PER_TILE
    def issue_ia(k, b):
        base = ebase + k * CHUNK
        pltpu.async_copy(ei_hbm.at[pl.ds(base, CHUNK)], srcv[b], sem_ia[b])
        pltpu.async_copy(attr_hbm.at[pl.ds(base, CHUNK)], attrv[b], sem_ia[b])

    def wait_ia(b):
        pltpu.make_async_copy(ei_hbm.at[pl.ds(0, CHUNK)], srcv[b],
                              sem_ia[b]).wait()
        pltpu.make_async_copy(attr_hbm.at[pl.ds(0, CHUNK)], attrv[b],
                              sem_ia[b]).wait()

    def issue_id(k, b):
        base = ebase + k * CHUNK
        pltpu.async_copy(ei_hbm.at[pl.ds(N_EDGES + base, CHUNK)],
                         dstv[b], sem_id[b])

    def wait_id(b):
        pltpu.make_async_copy(ei_hbm.at[pl.ds(0, CHUNK)], dstv[b],
                              sem_id[b]).wait()

    def issue_gather(b):
        pltpu.async_copy(x_hbm.at[srcv[b]], rows[b], sem_g[b])

    def wait_gather(b):
        pltpu.make_async_copy(x_hbm.at[srcv[b]], rows[b], sem_g[b]).wait()

    def issue_scatter(b):
        pltpu.async_copy(rows[b], acc.at[dstv[b]], sem_s[b], add=True)

    def wait_scatter(b):
        pltpu.make_async_copy(rows[b], acc.at[dstv[b]], sem_s[b]).wait()

    def multiply(b):
        def _group(g, carry):
            av = attrv[b][pl.ds(g * 16, 16)]
            for i in range(16):
                sc = _bcast_lane(av, i)
                e = g * 16 + i
                for j in range(NVEC):
                    sl = pl.ds(j * 16, 16)
                    rows[b][e, sl] = rows[b][e, sl] * sc
            return carry
        lax.fori_loop(0, CHUNK // 16, _group, 0)

    issue_ia(0, 0)
    issue_ia(1, 1)
    issue_ia(2, 2)
    issue_id(0, 0)

    def _zero_row(r, carry):
        for j in range(NVEC):
            rows[3][r, pl.ds(j * 16, 16)] = jnp.zeros((16,), jnp.float32)
        return carry
    lax.fori_loop(0, CHUNK, _zero_row, 0)
    for t in range(ROWS_PER_TILE // CHUNK):
        pltpu.sync_copy(rows[3], acc.at[pl.ds(rbase + t * CHUNK, CHUNK)])
    _rem = ROWS_PER_TILE % CHUNK
    if _rem:
        pltpu.sync_copy(
            rows[3].at[pl.ds(0, _rem)],
            acc.at[pl.ds(rbase + (ROWS_PER_TILE // CHUNK) * CHUNK, _rem)])

    @pl.when(s == 0)
    def _zero_tail():
        pltpu.sync_copy(rows[3].at[pl.ds(0, ROWS_TAIL)],
                        acc.at[pl.ds(NS * ROWS_PER_TILE, ROWS_TAIL)])

    wait_ia(0)
    issue_gather(0)
    wait_ia(1)
    issue_gather(1)
    plsc.subcore_barrier()

    def _super(it, carry):
        for r in range(NRING):
            k = it * NRING + r
            p = r
            @pl.when(k >= 2)
            def _w():
                wait_scatter((r + 2) % NRING)
            @pl.when(k + 3 < NCHUNKS)
            def _b1():
                issue_ia(k + 3, (r + 3) % NRING)
            @pl.when(k + 1 < NCHUNKS)
            def _b2():
                issue_id(k + 1, (r + 1) % NRING)
            @pl.when(k + 2 < NCHUNKS)
            def _f():
                wait_ia((r + 2) % NRING)
                issue_gather((r + 2) % NRING)
            wait_gather(p)
            multiply(p)
            wait_id(p)
            issue_scatter(p)
        return carry
    lax.fori_loop(0, NMAIN // NRING, _super, 0)

    wait_scatter(2)
    wait_gather(0)
    multiply(0)
    wait_id(0)
    issue_scatter(0)
    wait_scatter(3)
    wait_scatter(0)

    plsc.subcore_barrier()
    pltpu.sync_copy(acc.at[pl.ds(rbase, ROWS_PER_TILE)],
                    out_hbm.at[c, pl.ds(rbase, ROWS_PER_TILE)])

    @pl.when(s == 0)
    def _out_tail():
        pltpu.sync_copy(acc.at[pl.ds(NS * ROWS_PER_TILE, ROWS_TAIL)],
                        out_hbm.at[c, pl.ds(NS * ROWS_PER_TILE, ROWS_TAIL)])


@jax.jit
def _sc_agg(x, ei, attr):
    mesh = plsc.VectorSubcoreMesh(core_axis_name="c", subcore_axis_name="s")
    return pl.kernel(
        _sc_agg_body,
        out_type=jax.ShapeDtypeStruct((NC, N_NODES, F), jnp.float32),
        mesh=mesh,
        scratch_types=[
            pltpu.VMEM_SHARED((N_NODES, F), jnp.float32),
            [pltpu.VMEM((CHUNK,), jnp.int32) for _ in range(NRING)],
            [pltpu.VMEM((CHUNK,), jnp.int32) for _ in range(NRING)],
            [pltpu.VMEM((CHUNK,), jnp.float32) for _ in range(NRING)],
            [pltpu.VMEM((CHUNK, F), jnp.float32) for _ in range(NRING)],
            [pltpu.SemaphoreType.DMA for _ in range(NRING)],
            [pltpu.SemaphoreType.DMA for _ in range(NRING)],
            [pltpu.SemaphoreType.DMA for _ in range(NRING)],
            [pltpu.SemaphoreType.DMA for _ in range(NRING)],
        ],
    )(x, ei, attr)


ROWBLK = 2000
NBLK = N_NODES // ROWBLK


def _tc_body(se_ref, agg_ref, u_ref, wt, b2, k_ref, q_ref):
    xa = agg_ref[0] + agg_ref[1]
    ri = (lax.broadcasted_iota(jnp.int32, (ROWBLK, G), 0)
          + pl.program_id(0) * ROWBLK)
    oh = ((ri >= se_ref[0:1, :]) & (ri < se_ref[1:2, :])).astype(jnp.float32)
    hp = lax.Precision.HIGHEST
    uw = jnp.dot(u_ref[...], wt[F:, :], precision=hp)
    kq = (jnp.dot(xa, wt[:F, :], precision=hp)
          + jnp.dot(oh, uw, precision=hp) + b2[...])
    k_ref[...] = kq[:, :F]
    q_ref[...] = kq[:, F:]


@jax.jit
def _tc_linear(se, agg, u, wt, b2):
    full = lambda *shape: pl.BlockSpec(shape, lambda i: tuple(0 for _ in shape))
    return pl.pallas_call(
        _tc_body,
        grid=(NBLK,),
        in_specs=[
            full(2, G),
            pl.BlockSpec((NC, ROWBLK, F), lambda i: (0, i, 0)),
            full(G, F),
            full(2 * F, 2 * F),
            full(1, 2 * F),
        ],
        out_specs=[
            pl.BlockSpec((ROWBLK, F), lambda i: (i, 0)),
            pl.BlockSpec((ROWBLK, F), lambda i: (i, 0)),
        ],
        out_shape=[
            jax.ShapeDtypeStruct((N_NODES, F), jnp.float32),
            jax.ShapeDtypeStruct((N_NODES, F), jnp.float32),
        ],
    )(se, agg, u, wt, b2)


def kernel(x, edge_index, edge_attr, u, batch, WK, bK, WQ, bQ):
    ei = edge_index.astype(jnp.int32).reshape(2 * N_EDGES)
    attr = edge_attr.T.reshape(N_EDGES)
    agg = _sc_agg(x, ei, attr)
    b32 = batch.astype(jnp.int32)
    gids = jnp.arange(G, dtype=jnp.int32)
    starts = jnp.sum(b32[None, :] < gids[:, None], axis=1, dtype=jnp.int32)
    ends = jnp.concatenate([starts[1:], jnp.array([N_NODES], jnp.int32)])
    se = jnp.stack([starts, ends])
    wt = jnp.concatenate([WK, WQ], axis=0).T
    b2 = jnp.concatenate([bK, bQ]).reshape(1, 2 * F)
    K, Q = _tc_linear(se, agg, u, wt, b2)
    return K, Q
